# baseline (speedup 1.0000x reference)
import functools

import jax
import jax.numpy as jnp
from jax import lax
from jax.experimental import pallas as pl
from jax.experimental.pallas import tpu as pltpu
from jax.experimental.pallas import tpu_sc as plsc

N = 10000
E = 320000
D = 128

NC = 2
NS = 16
NW = NC * NS

N_PAD = 10240
ROWS_PER_TILE = N_PAD // NS
ZCHUNK = 128

WPT = 80
EROWS_P = NW * WPT
E_PAD = EROWS_P * D

_mesh = plsc.VectorSubcoreMesh(core_axis_name="c", subcore_axis_name="s")


@functools.partial(
    pl.kernel,
    mesh=_mesh,
    out_type=jax.ShapeDtypeStruct((NC, N_PAD, D), jnp.float32),
    scratch_types=[
        pltpu.VMEM((WPT, D), jnp.int32),
        pltpu.VMEM((D, D), jnp.float32),
        pltpu.VMEM((ZCHUNK, D), jnp.float32),
        pltpu.VMEM_SHARED((N_PAD, D), jnp.float32),
    ],
)
def _sc_degree(dst_hbm, deg_hbm, didx_c, ones_v, z_v, acc_sh):
    cid = lax.axis_index("c")
    sid = lax.axis_index("s")
    wid = cid * NS + sid

    @pl.loop(0, ZCHUNK)
    def _(r):
        @pl.loop(0, D, step=16)
        def _(cc):
            z_v[r, pl.ds(cc, 16)] = jnp.zeros((16,), jnp.float32)

    @pl.loop(0, D)
    def _(r):
        @pl.loop(0, D, step=16)
        def _(cc):
            ones_v[r, pl.ds(cc, 16)] = jnp.ones((16,), jnp.float32)

    pltpu.sync_copy(dst_hbm.at[pl.ds(wid * WPT, WPT)], didx_c)
    for j in range(ROWS_PER_TILE // ZCHUNK):
        pltpu.sync_copy(
            z_v, acc_sh.at[pl.ds(sid * ROWS_PER_TILE + j * ZCHUNK, ZCHUNK)]
        )
    plsc.subcore_barrier()

    @pl.loop(0, WPT)
    def _(w):
        pltpu.sync_copy(ones_v, acc_sh.at[didx_c.at[w]], add=True)

    plsc.subcore_barrier()
    for j in range(ROWS_PER_TILE // ZCHUNK):
        pltpu.sync_copy(
            acc_sh.at[pl.ds(sid * ROWS_PER_TILE + j * ZCHUNK, ZCHUNK)], z_v
        )
        pltpu.sync_copy(
            z_v, deg_hbm.at[cid, pl.ds(sid * ROWS_PER_TILE + j * ZCHUNK, ZCHUNK)]
        )


@functools.partial(
    pl.kernel,
    mesh=_mesh,
    out_type=jax.ShapeDtypeStruct((NC, N_PAD, D), jnp.float32),
    scratch_types=[
        pltpu.VMEM((WPT // 2, D), jnp.int32),
        pltpu.VMEM((WPT // 2, D), jnp.int32),
        pltpu.VMEM((D, D), jnp.float32),
        pltpu.VMEM((D, D), jnp.float32),
        pltpu.VMEM_SHARED((N_PAD, D), jnp.float32),
        pltpu.SemaphoreType.DMA,
        pltpu.SemaphoreType.DMA,
    ],
)
def _sc_scatter(g_hbm, src_hbm, dst_hbm, out_hbm, sidx_c, didx_c, rows0, rows1,
                acc_sh, sem0, sem1):
    cid = lax.axis_index("c")
    sid = lax.axis_index("s")
    wid = cid * NS + sid

    @pl.loop(0, ZCHUNK)
    def _(r):
        @pl.loop(0, D, step=16)
        def _(cc):
            rows0[r, pl.ds(cc, 16)] = jnp.zeros((16,), jnp.float32)

    for j in range(ROWS_PER_TILE // ZCHUNK):
        pltpu.sync_copy(
            rows0, acc_sh.at[pl.ds(sid * ROWS_PER_TILE + j * ZCHUNK, ZCHUNK)]
        )
    plsc.subcore_barrier()

    HW = WPT // 2
    for half in range(2):
        cbase = wid * WPT + half * HW
        pltpu.sync_copy(src_hbm.at[pl.ds(cbase, HW)], sidx_c)
        pltpu.sync_copy(dst_hbm.at[pl.ds(cbase, HW)], didx_c)

        pltpu.async_copy(g_hbm.at[sidx_c.at[0]], rows0, sem0)
        pltpu.async_copy(g_hbm.at[sidx_c.at[1]], rows1, sem1)

        @pl.loop(0, HW, step=2)
        def _(w):
            pltpu.make_async_copy(g_hbm.at[pl.ds(0, D)], rows0, sem0).wait()
            pltpu.sync_copy(rows0, acc_sh.at[didx_c.at[w]], add=True)

            @pl.when(w + 2 < HW)
            def _():
                pltpu.async_copy(g_hbm.at[sidx_c.at[w + 2]], rows0, sem0)

            pltpu.make_async_copy(g_hbm.at[pl.ds(0, D)], rows1, sem1).wait()
            pltpu.sync_copy(rows1, acc_sh.at[didx_c.at[w + 1]], add=True)

            @pl.when(w + 3 < HW)
            def _():
                pltpu.async_copy(g_hbm.at[sidx_c.at[w + 3]], rows1, sem1)

    plsc.subcore_barrier()
    for j in range(ROWS_PER_TILE // ZCHUNK):
        pltpu.sync_copy(
            acc_sh.at[pl.ds(sid * ROWS_PER_TILE + j * ZCHUNK, ZCHUNK)], rows0
        )
        pltpu.sync_copy(
            rows0, out_hbm.at[cid, pl.ds(sid * ROWS_PER_TILE + j * ZCHUNK, ZCHUNK)]
        )


_RB = 1000
_NB = N // _RB


def _tc_matmul_body(x_ref, w_ref, h_ref):
    h_ref[...] = lax.dot_general(
        x_ref[...], w_ref[...], (((1,), (1,)), ((), ())),
        preferred_element_type=jnp.float32,
    )


def _tc_scale_body(h_ref, deg_ref, g_ref):
    deg = deg_ref[0, :, 0:1] + deg_ref[1, :, 0:1] + 1.0
    g_ref[...] = h_ref[...] * lax.rsqrt(deg)


def _tc_combine_body(sp_ref, g_ref, deg_ref, b_ref, t_ref, ps_ref, pss_ref):
    deg = deg_ref[0, :, 0:1] + deg_ref[1, :, 0:1] + 1.0
    dinv = lax.rsqrt(deg)
    t = (sp_ref[0] + sp_ref[1] + g_ref[...]) * dinv + b_ref[...]
    t_ref[...] = t
    ps_ref[...] = jnp.sum(t, axis=0, keepdims=True)[None]
    pss_ref[...] = jnp.sum(t * t, axis=0, keepdims=True)[None]


def _tc_finish_body(t_ref, ps_ref, pss_ref, gamma_ref, beta_ref, x_ref, y_ref):
    inv_n = 1.0 / N
    mean = jnp.sum(ps_ref[...], axis=0) * inv_n
    var = jnp.sum(pss_ref[...], axis=0) * inv_n - mean * mean
    scale = gamma_ref[...] * lax.rsqrt(var + 1e-5)
    bn = (t_ref[...] - mean) * scale + beta_ref[...]
    y_ref[...] = jnp.maximum(bn, 0.0) + x_ref[...]


@jax.jit
def kernel(x, edge_index, W, b, gamma, beta):
    pad_n = E_PAD - E
    pad_src = jnp.zeros((pad_n,), jnp.int32)
    pad_dst = N + (jnp.arange(pad_n, dtype=jnp.int32) % (N_PAD - N))
    src2 = jnp.concatenate([edge_index[0], pad_src]).reshape(EROWS_P, D)
    dst2 = jnp.concatenate([edge_index[1], pad_dst]).reshape(EROWS_P, D)
    b2 = b.reshape(1, D)
    gamma2 = gamma.reshape(1, D)
    beta2 = beta.reshape(1, D)

    deg_p = _sc_degree(dst2)

    h = pl.pallas_call(
        _tc_matmul_body,
        grid=(_NB,),
        in_specs=[
            pl.BlockSpec((_RB, D), lambda i: (i, 0)),
            pl.BlockSpec((D, D), lambda i: (0, 0)),
        ],
        out_specs=pl.BlockSpec((_RB, D), lambda i: (i, 0)),
        out_shape=jax.ShapeDtypeStruct((N, D), jnp.float32),
    )(x, W)

    g = pl.pallas_call(
        _tc_scale_body,
        grid=(_NB,),
        in_specs=[
            pl.BlockSpec((_RB, D), lambda i: (i, 0)),
            pl.BlockSpec((NC, _RB, D), lambda i: (0, i, 0)),
        ],
        out_specs=pl.BlockSpec((_RB, D), lambda i: (i, 0)),
        out_shape=jax.ShapeDtypeStruct((N, D), jnp.float32),
    )(h, deg_p)

    s_p = _sc_scatter(g, src2, dst2)

    t, ps, pss = pl.pallas_call(
        _tc_combine_body,
        grid=(_NB,),
        in_specs=[
            pl.BlockSpec((NC, _RB, D), lambda i: (0, i, 0)),
            pl.BlockSpec((_RB, D), lambda i: (i, 0)),
            pl.BlockSpec((NC, _RB, D), lambda i: (0, i, 0)),
            pl.BlockSpec((1, D), lambda i: (0, 0)),
        ],
        out_specs=[
            pl.BlockSpec((_RB, D), lambda i: (i, 0)),
            pl.BlockSpec((1, 1, D), lambda i: (i, 0, 0)),
            pl.BlockSpec((1, 1, D), lambda i: (i, 0, 0)),
        ],
        out_shape=[
            jax.ShapeDtypeStruct((N, D), jnp.float32),
            jax.ShapeDtypeStruct((_NB, 1, D), jnp.float32),
            jax.ShapeDtypeStruct((_NB, 1, D), jnp.float32),
        ],
    )(s_p, g, deg_p, b2)

    y = pl.pallas_call(
        _tc_finish_body,
        grid=(_NB,),
        in_specs=[
            pl.BlockSpec((_RB, D), lambda i: (i, 0)),
            pl.BlockSpec((_NB, 1, D), lambda i: (0, 0, 0)),
            pl.BlockSpec((_NB, 1, D), lambda i: (0, 0, 0)),
            pl.BlockSpec((1, D), lambda i: (0, 0)),
            pl.BlockSpec((1, D), lambda i: (0, 0)),
            pl.BlockSpec((_RB, D), lambda i: (i, 0)),
        ],
        out_specs=pl.BlockSpec((_RB, D), lambda i: (i, 0)),
        out_shape=jax.ShapeDtypeStruct((N, D), jnp.float32),
    )(t, ps, pss, gamma2, beta2, x)

    return y


# speedup vs baseline: 2.2287x; 2.2287x over previous
import functools

import jax
import jax.numpy as jnp
from jax import lax
from jax.experimental import pallas as pl
from jax.experimental.pallas import tpu as pltpu
from jax.experimental.pallas import tpu_sc as plsc

N = 10000
E = 320000
D = 128

NC = 2
NS = 16
NW = NC * NS

N_PAD = 10240
ROWS_PER_TILE = N_PAD // NS
ZCHUNK = 128

WPT = 80
EROWS_P = NW * WPT
E_PAD = EROWS_P * D

_mesh = plsc.VectorSubcoreMesh(core_axis_name="c", subcore_axis_name="s")


@functools.partial(
    pl.kernel,
    mesh=_mesh,
    out_type=jax.ShapeDtypeStruct((NC, N_PAD, D), jnp.float32),
    scratch_types=[
        pltpu.VMEM((WPT, D), jnp.int32),
        pltpu.VMEM((D, D), jnp.float32),
        pltpu.VMEM((ZCHUNK, D), jnp.float32),
        pltpu.VMEM_SHARED((N_PAD, D), jnp.float32),
    ],
)
def _sc_degree(dst_hbm, deg_hbm, didx_c, ones_v, z_v, acc_sh):
    cid = lax.axis_index("c")
    sid = lax.axis_index("s")
    wid = cid * NS + sid

    @pl.loop(0, ZCHUNK)
    def _(r):
        @pl.loop(0, D, step=16)
        def _(cc):
            z_v[r, pl.ds(cc, 16)] = jnp.zeros((16,), jnp.float32)

    @pl.loop(0, D)
    def _(r):
        @pl.loop(0, D, step=16)
        def _(cc):
            ones_v[r, pl.ds(cc, 16)] = jnp.ones((16,), jnp.float32)

    pltpu.sync_copy(dst_hbm.at[pl.ds(wid * WPT, WPT)], didx_c)
    for j in range(ROWS_PER_TILE // ZCHUNK):
        pltpu.sync_copy(
            z_v, acc_sh.at[pl.ds(sid * ROWS_PER_TILE + j * ZCHUNK, ZCHUNK)]
        )
    plsc.subcore_barrier()

    @pl.loop(0, WPT)
    def _(w):
        pltpu.sync_copy(ones_v, acc_sh.at[didx_c.at[w]], add=True)

    plsc.subcore_barrier()
    for j in range(ROWS_PER_TILE // ZCHUNK):
        pltpu.sync_copy(
            acc_sh.at[pl.ds(sid * ROWS_PER_TILE + j * ZCHUNK, ZCHUNK)], z_v
        )
        pltpu.sync_copy(
            z_v, deg_hbm.at[cid, pl.ds(sid * ROWS_PER_TILE + j * ZCHUNK, ZCHUNK)]
        )


@functools.partial(
    pl.kernel,
    mesh=_mesh,
    out_type=jax.ShapeDtypeStruct((NC, N_PAD, D), jnp.float32),
    scratch_types=[
        pltpu.VMEM((WPT // 2, D), jnp.int32),
        pltpu.VMEM((WPT // 2, D), jnp.int32),
        pltpu.VMEM((D, D), jnp.float32),
        pltpu.VMEM((D, D), jnp.float32),
        pltpu.VMEM_SHARED((N_PAD, D), jnp.float32),
        pltpu.SemaphoreType.DMA,
        pltpu.SemaphoreType.DMA,
    ],
)
def _sc_scatter(g_hbm, src_hbm, dst_hbm, out_hbm, sidx_c, didx_c, rows0, rows1,
                acc_sh, sem0, sem1):
    cid = lax.axis_index("c")
    sid = lax.axis_index("s")
    wid = cid * NS + sid

    @pl.loop(0, ZCHUNK)
    def _(r):
        @pl.loop(0, D, step=16)
        def _(cc):
            rows0[r, pl.ds(cc, 16)] = jnp.zeros((16,), jnp.float32)

    for j in range(ROWS_PER_TILE // ZCHUNK):
        pltpu.sync_copy(
            rows0, acc_sh.at[pl.ds(sid * ROWS_PER_TILE + j * ZCHUNK, ZCHUNK)]
        )
    plsc.subcore_barrier()

    HW = WPT // 2
    for half in range(2):
        cbase = wid * WPT + half * HW
        pltpu.sync_copy(src_hbm.at[pl.ds(cbase, HW)], sidx_c)
        pltpu.sync_copy(dst_hbm.at[pl.ds(cbase, HW)], didx_c)

        pltpu.async_copy(g_hbm.at[sidx_c.at[0]], rows0, sem0)
        pltpu.async_copy(g_hbm.at[sidx_c.at[1]], rows1, sem1)

        @pl.loop(0, HW, step=2)
        def _(w):
            pltpu.make_async_copy(g_hbm.at[pl.ds(0, D)], rows0, sem0).wait()
            pltpu.sync_copy(rows0, acc_sh.at[didx_c.at[w]], add=True)

            @pl.when(w + 2 < HW)
            def _():
                pltpu.async_copy(g_hbm.at[sidx_c.at[w + 2]], rows0, sem0)

            pltpu.make_async_copy(g_hbm.at[pl.ds(0, D)], rows1, sem1).wait()
            pltpu.sync_copy(rows1, acc_sh.at[didx_c.at[w + 1]], add=True)

            @pl.when(w + 3 < HW)
            def _():
                pltpu.async_copy(g_hbm.at[sidx_c.at[w + 3]], rows1, sem1)

    plsc.subcore_barrier()
    for j in range(ROWS_PER_TILE // ZCHUNK):
        pltpu.sync_copy(
            acc_sh.at[pl.ds(sid * ROWS_PER_TILE + j * ZCHUNK, ZCHUNK)], rows0
        )
        pltpu.sync_copy(
            rows0, out_hbm.at[cid, pl.ds(sid * ROWS_PER_TILE + j * ZCHUNK, ZCHUNK)]
        )


_RB = 1000
_NB = N // _RB


def _tc_matmul_body(x_ref, w_ref, h_ref):
    h_ref[...] = lax.dot_general(
        x_ref[...], w_ref[...], (((1,), (1,)), ((), ())),
        preferred_element_type=jnp.float32,
    )


def _tc_scale_body(h_ref, deg_ref, g_ref):
    deg = deg_ref[0, :, 0:1] + deg_ref[1, :, 0:1] + 1.0
    g_ref[...] = h_ref[...] * lax.rsqrt(deg)


def _tc_combine_body(sp_ref, g_ref, deg_ref, b_ref, t_ref, ps_ref, pss_ref):
    deg = deg_ref[0, :, 0:1] + deg_ref[1, :, 0:1] + 1.0
    dinv = lax.rsqrt(deg)
    t = (sp_ref[0] + sp_ref[1] + g_ref[...]) * dinv + b_ref[...]
    t_ref[...] = t
    ps_ref[...] = jnp.sum(t, axis=0, keepdims=True)[None]
    pss_ref[...] = jnp.sum(t * t, axis=0, keepdims=True)[None]


def _tc_finish_body(t_ref, ps_ref, pss_ref, gamma_ref, beta_ref, x_ref, y_ref):
    inv_n = 1.0 / N
    mean = jnp.sum(ps_ref[...], axis=0) * inv_n
    var = jnp.sum(pss_ref[...], axis=0) * inv_n - mean * mean
    scale = gamma_ref[...] * lax.rsqrt(var + 1e-5)
    bn = (t_ref[...] - mean) * scale + beta_ref[...]
    y_ref[...] = jnp.maximum(bn, 0.0) + x_ref[...]


@jax.jit
def kernel(x, edge_index, W, b, gamma, beta):
    pad_n = E_PAD - E
    pad_src = jnp.arange(pad_n, dtype=jnp.int32) % N
    pad_dst = N + (jnp.arange(pad_n, dtype=jnp.int32) % (N_PAD - N))
    src2 = jnp.concatenate([edge_index[0], pad_src]).reshape(EROWS_P, D)
    dst2 = jnp.concatenate([edge_index[1], pad_dst]).reshape(EROWS_P, D)
    b2 = b.reshape(1, D)
    gamma2 = gamma.reshape(1, D)
    beta2 = beta.reshape(1, D)

    deg_p = _sc_degree(dst2)

    h = pl.pallas_call(
        _tc_matmul_body,
        grid=(_NB,),
        in_specs=[
            pl.BlockSpec((_RB, D), lambda i: (i, 0)),
            pl.BlockSpec((D, D), lambda i: (0, 0)),
        ],
        out_specs=pl.BlockSpec((_RB, D), lambda i: (i, 0)),
        out_shape=jax.ShapeDtypeStruct((N, D), jnp.float32),
    )(x, W)

    g = pl.pallas_call(
        _tc_scale_body,
        grid=(_NB,),
        in_specs=[
            pl.BlockSpec((_RB, D), lambda i: (i, 0)),
            pl.BlockSpec((NC, _RB, D), lambda i: (0, i, 0)),
        ],
        out_specs=pl.BlockSpec((_RB, D), lambda i: (i, 0)),
        out_shape=jax.ShapeDtypeStruct((N, D), jnp.float32),
    )(h, deg_p)

    s_p = _sc_scatter(g, src2, dst2)

    t, ps, pss = pl.pallas_call(
        _tc_combine_body,
        grid=(_NB,),
        in_specs=[
            pl.BlockSpec((NC, _RB, D), lambda i: (0, i, 0)),
            pl.BlockSpec((_RB, D), lambda i: (i, 0)),
            pl.BlockSpec((NC, _RB, D), lambda i: (0, i, 0)),
            pl.BlockSpec((1, D), lambda i: (0, 0)),
        ],
        out_specs=[
            pl.BlockSpec((_RB, D), lambda i: (i, 0)),
            pl.BlockSpec((1, 1, D), lambda i: (i, 0, 0)),
            pl.BlockSpec((1, 1, D), lambda i: (i, 0, 0)),
        ],
        out_shape=[
            jax.ShapeDtypeStruct((N, D), jnp.float32),
            jax.ShapeDtypeStruct((_NB, 1, D), jnp.float32),
            jax.ShapeDtypeStruct((_NB, 1, D), jnp.float32),
        ],
    )(s_p, g, deg_p, b2)

    y = pl.pallas_call(
        _tc_finish_body,
        grid=(_NB,),
        in_specs=[
            pl.BlockSpec((_RB, D), lambda i: (i, 0)),
            pl.BlockSpec((_NB, 1, D), lambda i: (0, 0, 0)),
            pl.BlockSpec((_NB, 1, D), lambda i: (0, 0, 0)),
            pl.BlockSpec((1, D), lambda i: (0, 0)),
            pl.BlockSpec((1, D), lambda i: (0, 0)),
            pl.BlockSpec((_RB, D), lambda i: (i, 0)),
        ],
        out_specs=pl.BlockSpec((_RB, D), lambda i: (i, 0)),
        out_shape=jax.ShapeDtypeStruct((N, D), jnp.float32),
    )(t, ps, pss, gamma2, beta2, x)

    return y
